# baseline (speedup 1.0000x reference)
import jax
import jax.numpy as jnp
from jax import lax
from jax.experimental import pallas as pl
from jax.experimental.pallas import tpu as pltpu
from jax.experimental.pallas import tpu_sc as plsc

F = 16
B = 16384
NW = 32
BPW = B // NW
CHUNK = 128
NCHUNK = BPW // CHUNK
GROUPS = BPW // F


def _body(users, pos_items, neg_items, ue, ie, ub, ib, out,
          idx_u, idx_p, idx_n, rows_u, rows_p, rows_n, bu, bp, bn, out_v,
          sem):
  wid = lax.axis_index("c") * 16 + lax.axis_index("s")
  base = wid * BPW

  pltpu.sync_copy(users.at[pl.ds(base, BPW)], idx_u)
  pltpu.sync_copy(pos_items.at[pl.ds(base, BPW)], idx_p)
  pltpu.sync_copy(neg_items.at[pl.ds(base, BPW)], idx_n)

  copies = []
  for idx, table, dst in ((idx_u, ue, rows_u), (idx_p, ie, rows_p),
                          (idx_n, ie, rows_n), (idx_u, ub, bu),
                          (idx_p, ib, bp), (idx_n, ib, bn)):
    for j in range(NCHUNK):
      sl = pl.ds(j * CHUNK, CHUNK)
      copies.append(pltpu.async_copy(table.at[idx.at[sl]], dst.at[sl], sem))
  for c in copies:
    c.wait()

  lanes = lax.iota(jnp.int32, F)

  def group(g, carry):
    bidx = g * F + lanes
    acc = bu[pl.ds(g * F, F)] + bp[pl.ds(g * F, F)] - bn[pl.ds(g * F, F)]
    for f in range(F):
      fvec = jnp.full((F,), f, jnp.int32)
      u = plsc.load_gather(rows_u, [bidx, fvec])
      p = plsc.load_gather(rows_p, [bidx, fvec])
      n = plsc.load_gather(rows_n, [bidx, fvec])
      acc = acc + u * (p - n)
    out_v[pl.ds(g * F, F)] = acc
    return carry

  lax.fori_loop(0, GROUPS, group, 0)
  pltpu.sync_copy(out_v, out.at[pl.ds(base, BPW)])


@jax.jit
def kernel(users, pos_items, neg_items, user_embeddings, item_embeddings,
           user_biases, item_biases):
  mesh = plsc.VectorSubcoreMesh(core_axis_name="c", subcore_axis_name="s")
  run = pl.kernel(
      _body,
      out_type=jax.ShapeDtypeStruct((B,), jnp.float32),
      mesh=mesh,
      scratch_types=[
          pltpu.VMEM((BPW,), jnp.int32),
          pltpu.VMEM((BPW,), jnp.int32),
          pltpu.VMEM((BPW,), jnp.int32),
          pltpu.VMEM((BPW, F), jnp.float32),
          pltpu.VMEM((BPW, F), jnp.float32),
          pltpu.VMEM((BPW, F), jnp.float32),
          pltpu.VMEM((BPW,), jnp.float32),
          pltpu.VMEM((BPW,), jnp.float32),
          pltpu.VMEM((BPW,), jnp.float32),
          pltpu.VMEM((BPW,), jnp.float32),
          pltpu.SemaphoreType.DMA,
      ],
      compiler_params=pltpu.CompilerParams(needs_layout_passes=False,
                                           use_tc_tiling_on_sc=False),
  )
  return run(users.astype(jnp.int32), pos_items.astype(jnp.int32),
             neg_items.astype(jnp.int32), user_embeddings, item_embeddings,
             user_biases.reshape(-1), item_biases.reshape(-1))
